# baseline (speedup 1.0000x reference)
import math

import jax
import jax.numpy as jnp
from jax.experimental import pallas as pl
from jax.experimental.pallas import tpu as pltpu

B, S, H, M, LC, NL, I, NH = 16, 512, 768, 32, 32, 7, 3072, 8
DH = H // NH
DP = 128
HP = NH * DP
LN_EPS = 1e-12
MLC = M * LC

_f32 = jnp.float32
_bf16 = jnp.bfloat16
_CDIM0 = (((0,), (0,)), ((), ()))


def _dot(a, b):
  return jnp.dot(a, b, preferred_element_type=_f32)


def _gcn_kernel(enc_ref, wr_ref, wrm_ref, wrmm_ref, adj_ref,
                gc1_w_ref, gc1_b_ref, gc2_w_ref, gc2_b_ref,
                proj_w_ref, proj_b_ref,
                cv_ref,
                ohm_s, xs, s1, s2, ms, pooled_s):
  wr = wr_ref[0]
  wrm = wrm_ref[0]
  iota_s = jax.lax.broadcasted_iota(jnp.int32, (S, MLC), 0)
  ohm_s[...] = jnp.where(iota_s == wr,
                         jnp.broadcast_to(wrm, (S, MLC)), 0.0).astype(_bf16)

  xs[...] = jax.lax.dot_general(ohm_s[...], enc_ref[0], _CDIM0,
                                preferred_element_type=_f32).astype(_bf16)
  ones = jnp.ones((S, 128), _bf16)
  ms[...] = jax.lax.dot_general(ohm_s[...], ones, _CDIM0,
                                preferred_element_type=_f32)

  s1[...] = _dot(xs[...], gc1_w_ref[...]).astype(_bf16)
  b1 = gc1_b_ref[...]
  for m in range(M):
    sl = slice(m * LC, (m + 1) * LC)
    a_m = adj_ref[0, m].astype(_bf16)
    s2[sl, :] = jnp.maximum(_dot(a_m, s1[sl, :]) + b1, 0.0).astype(_bf16)

  s1[...] = _dot(s2[...], gc2_w_ref[...]).astype(_bf16)
  b2 = gc2_b_ref[...]
  for m in range(M):
    sl = slice(m * LC, (m + 1) * LC)
    a_m = adj_ref[0, m].astype(_bf16)
    h2m = jnp.maximum(_dot(a_m, s1[sl, :]) + b2, 0.0)
    h2m = h2m * pltpu.repeat(ms[sl, :], H // 128, axis=1)
    xm = xs[sl, :].astype(_f32)
    row = jnp.concatenate([
        jnp.max(xm, axis=0, keepdims=True),
        jnp.max(h2m, axis=0, keepdims=True),
        jnp.sum(xm, axis=0, keepdims=True),
        jnp.sum(h2m, axis=0, keepdims=True),
    ], axis=1)
    pooled_s[m:m + 1, :] = row

  lens = jnp.sum(wrmm_ref[0], axis=1, keepdims=True) + 1e-45
  pooled = pooled_s[...]
  pooled = jnp.concatenate([pooled[:, :2 * H], pooled[:, 2 * H:] / lens],
                           axis=1).astype(_bf16)
  cv_ref[0] = jnp.maximum(_dot(pooled, proj_w_ref[...]) + proj_b_ref[...],
                          0.0)


def _attn_kernel(cv_ref, qw_ref, qb_ref, kw_ref, kb_ref, vw_ref, vb_ref,
                 aow_ref, aob_ref, ln1g_ref, ln1b_ref, cnm_ref,
                 attn_ref, qs, ks, vs, ctx_s):
  cv = cv_ref[...]
  rows = cv.shape[0]
  nb = rows // M
  cvb = cv.astype(_bf16)
  qs[...] = (_dot(cvb, qw_ref[...]) + qb_ref[...]).astype(_bf16)
  ks[...] = (_dot(cvb, kw_ref[...]) + kb_ref[...]).astype(_bf16)
  vs[...] = (_dot(cvb, vw_ref[...]) + vb_ref[...]).astype(_bf16)
  scale = 1.0 / math.sqrt(DH)
  for bb in range(nb):
    sl = slice(bb * M, (bb + 1) * M)
    amask = (1.0 - cnm_ref[bb]) * (-10000.0)
    qb_ = qs[sl, :]
    kb_ = ks[sl, :]
    vb_ = vs[sl, :]
    parts = []
    for h in range(NH):
      hs = slice(h * DP, (h + 1) * DP)
      sc = jax.lax.dot_general(qb_[:, hs], kb_[:, hs],
                               (((1,), (1,)), ((), ())),
                               preferred_element_type=_f32) * scale + amask
      sc = sc - jnp.max(sc, axis=1, keepdims=True)
      e = jnp.exp(sc)
      att = e / jnp.sum(e, axis=1, keepdims=True)
      parts.append(_dot(att.astype(_bf16), vb_[:, hs]))
    ctx_s[sl, :] = jnp.concatenate(parts, axis=1).astype(_bf16)
  co = _dot(ctx_s[...], aow_ref[...]) + aob_ref[...] + cv
  mu = jnp.mean(co, axis=1, keepdims=True)
  d = co - mu
  var = jnp.mean(d * d, axis=1, keepdims=True)
  attn_ref[...] = (d * jax.lax.rsqrt(var + LN_EPS) * ln1g_ref[...]
                   + ln1b_ref[...])


def _ffn_kernel(attn_ref, intw_ref, intb_ref, outw_ref, outb_ref,
                ln2g_ref, ln2b_ref, decw_ref, decb_ref, tgt_ref, cnmc_ref,
                kl_ref, cn_ref, inter_s):
  attn = attn_ref[...]
  rows = attn.shape[0]
  z = _dot(attn.astype(_bf16), intw_ref[...]) + intb_ref[...]
  gelu = z * 0.5 * (1.0 + jax.lax.erf(z * (1.0 / math.sqrt(2.0))))
  inter_s[...] = gelu.astype(_bf16)
  o = _dot(inter_s[...], outw_ref[...]) + outb_ref[...] + attn
  mu = jnp.mean(o, axis=1, keepdims=True)
  d = o - mu
  var = jnp.mean(d * d, axis=1, keepdims=True)
  out = d * jax.lax.rsqrt(var + LN_EPS) * ln2g_ref[...] + ln2b_ref[...]
  pred = _dot(out, decw_ref[...]) + decb_ref[...]
  mx = jnp.max(pred, axis=1, keepdims=True)
  e = jnp.exp(pred - mx)
  lse = jnp.log(jnp.sum(e, axis=1, keepdims=True)) + mx
  logp = pred - lse
  t = tgt_ref[...].reshape(rows, NL)
  kl_el = jnp.where(t > 0, t * jnp.log(jnp.where(t > 0, t, 1.0)), 0.0) \
      - t * logp
  klc = jnp.sum(kl_el, axis=1, keepdims=True) * (1.0 / NL)
  cnm = cnmc_ref[...].reshape(rows, 1)
  kl_sum = jnp.sum(klc * cnm, axis=0, keepdims=True)
  cn_sum = jnp.sum(cnm, axis=0, keepdims=True)
  kl_ref[...] = jnp.broadcast_to(kl_sum.reshape(1, 1, 1), (1, 1, 128))
  cn_ref[...] = jnp.broadcast_to(cn_sum.reshape(1, 1, 1), (1, 1, 128))


def _loss_kernel(kl_ref, cn_ref, out_ref):
  kl = jnp.sum(kl_ref[:, 0, 0:1], axis=0, keepdims=True)
  cn = jnp.sum(cn_ref[:, 0, 0:1], axis=0, keepdims=True)
  out_ref[...] = kl / cn


@jax.jit
def kernel(encoder_hs, word_recovery, word_recovery_mask, clause_num_mask,
           adj_matrix, target_labels,
           gc1_w, gc1_b, gc2_w, gc2_b, proj_w, proj_b,
           q_w, q_b, k_w, k_b, v_w, v_b, ao_w, ao_b, ln1_g, ln1_b,
           int_w, int_b, out_w, out_b, ln2_g, ln2_b, dec_w, dec_b):
  wr_flat = word_recovery.reshape(B, 1, MLC)
  wrm_row = word_recovery_mask.astype(_f32).reshape(B, 1, MLC)
  wrm_mat = word_recovery_mask.astype(_f32)
  cnm_row = clause_num_mask.astype(_f32).reshape(B, 1, M)
  cnm_col = clause_num_mask.astype(_f32).reshape(B, M, 1)
  enc_bf = encoder_hs.astype(_bf16)

  def pad_heads_cols(w):
    return jnp.pad(w.reshape(H, NH, DH), ((0, 0), (0, 0), (0, DP - DH))
                   ).reshape(H, HP)

  def pad_heads_bias(b):
    return jnp.pad(b.reshape(NH, DH), ((0, 0), (0, DP - DH))).reshape(1, HP)

  qw_p = pad_heads_cols(q_w).astype(_bf16)
  kw_p = pad_heads_cols(k_w).astype(_bf16)
  vw_p = pad_heads_cols(v_w).astype(_bf16)
  aow_p = jnp.pad(ao_w.reshape(NH, DH, H), ((0, 0), (0, DP - DH), (0, 0))
                  ).reshape(HP, H).astype(_bf16)
  qb_p = pad_heads_bias(q_b)
  kb_p = pad_heads_bias(k_b)
  vb_p = pad_heads_bias(v_b)

  row2 = lambda x: x.reshape(1, -1)
  const2 = lambda b: (0, 0)

  cv = pl.pallas_call(
      _gcn_kernel,
      grid=(B,),
      in_specs=[
          pl.BlockSpec((1, S, H), lambda b: (b, 0, 0)),
          pl.BlockSpec((1, 1, MLC), lambda b: (b, 0, 0)),
          pl.BlockSpec((1, 1, MLC), lambda b: (b, 0, 0)),
          pl.BlockSpec((1, M, LC), lambda b: (b, 0, 0)),
          pl.BlockSpec((1, M, LC, LC), lambda b: (b, 0, 0, 0)),
          pl.BlockSpec((H, H), const2),
          pl.BlockSpec((1, H), const2),
          pl.BlockSpec((H, H), const2),
          pl.BlockSpec((1, H), const2),
          pl.BlockSpec((4 * H, H), const2),
          pl.BlockSpec((1, H), const2),
      ],
      out_specs=pl.BlockSpec((1, M, H), lambda b: (b, 0, 0)),
      out_shape=jax.ShapeDtypeStruct((B, M, H), _f32),
      scratch_shapes=[
          pltpu.VMEM((S, MLC), _bf16),
          pltpu.VMEM((MLC, H), _bf16),
          pltpu.VMEM((MLC, H), _bf16),
          pltpu.VMEM((MLC, H), _bf16),
          pltpu.VMEM((MLC, 128), _f32),
          pltpu.VMEM((M, 4 * H), _f32),
      ],
      compiler_params=pltpu.CompilerParams(
          dimension_semantics=("parallel",),
          vmem_limit_bytes=56 * 1024 * 1024,
      ),
  )(enc_bf, wr_flat, wrm_row, wrm_mat, adj_matrix,
    gc1_w.astype(_bf16), row2(gc1_b), gc2_w.astype(_bf16), row2(gc2_b),
    proj_w.astype(_bf16), row2(proj_b))

  cv2 = cv.reshape(B * M, H)
  RB = B // 2
  R = RB * M

  attn = pl.pallas_call(
      _attn_kernel,
      grid=(2,),
      in_specs=[
          pl.BlockSpec((R, H), lambda c: (c, 0)),
          pl.BlockSpec((H, HP), const2),
          pl.BlockSpec((1, HP), const2),
          pl.BlockSpec((H, HP), const2),
          pl.BlockSpec((1, HP), const2),
          pl.BlockSpec((H, HP), const2),
          pl.BlockSpec((1, HP), const2),
          pl.BlockSpec((HP, H), const2),
          pl.BlockSpec((1, H), const2),
          pl.BlockSpec((1, H), const2),
          pl.BlockSpec((1, H), const2),
          pl.BlockSpec((RB, 1, M), lambda c: (c, 0, 0)),
      ],
      out_specs=pl.BlockSpec((R, H), lambda c: (c, 0)),
      out_shape=jax.ShapeDtypeStruct((B * M, H), _f32),
      scratch_shapes=[
          pltpu.VMEM((R, HP), _bf16),
          pltpu.VMEM((R, HP), _bf16),
          pltpu.VMEM((R, HP), _bf16),
          pltpu.VMEM((R, HP), _bf16),
      ],
      compiler_params=pltpu.CompilerParams(
          dimension_semantics=("parallel",),
          vmem_limit_bytes=56 * 1024 * 1024,
      ),
  )(cv2, qw_p, qb_p, kw_p, kb_p, vw_p, vb_p,
    aow_p, row2(ao_b), row2(ln1_g), row2(ln1_b), cnm_row)

  kl_parts, cn_parts = pl.pallas_call(
      _ffn_kernel,
      grid=(2,),
      in_specs=[
          pl.BlockSpec((R, H), lambda c: (c, 0)),
          pl.BlockSpec((H, I), const2),
          pl.BlockSpec((1, I), const2),
          pl.BlockSpec((I, H), const2),
          pl.BlockSpec((1, H), const2),
          pl.BlockSpec((1, H), const2),
          pl.BlockSpec((1, H), const2),
          pl.BlockSpec((H, NL), const2),
          pl.BlockSpec((1, NL), const2),
          pl.BlockSpec((RB, M, NL), lambda c: (c, 0, 0)),
          pl.BlockSpec((RB, M, 1), lambda c: (c, 0, 0)),
      ],
      out_specs=[
          pl.BlockSpec((1, 1, 128), lambda c: (c, 0, 0)),
          pl.BlockSpec((1, 1, 128), lambda c: (c, 0, 0)),
      ],
      out_shape=[
          jax.ShapeDtypeStruct((2, 1, 128), _f32),
          jax.ShapeDtypeStruct((2, 1, 128), _f32),
      ],
      scratch_shapes=[
          pltpu.VMEM((R, I), _bf16),
      ],
      compiler_params=pltpu.CompilerParams(
          dimension_semantics=("parallel",),
          vmem_limit_bytes=56 * 1024 * 1024,
      ),
  )(attn, int_w.astype(_bf16), row2(int_b), out_w.astype(_bf16),
    row2(out_b), row2(ln2_g), row2(ln2_b),
    dec_w, row2(dec_b), target_labels, cnm_col)

  loss = pl.pallas_call(
      _loss_kernel,
      out_shape=jax.ShapeDtypeStruct((1, 1), _f32),
  )(kl_parts, cn_parts)
  return loss.reshape(())


# speedup vs baseline: 1.3892x; 1.3892x over previous
import math

import jax
import jax.numpy as jnp
from jax.experimental import pallas as pl
from jax.experimental.pallas import tpu as pltpu

B, S, H, M, LC, NL, I, NH = 16, 512, 768, 32, 32, 7, 3072, 8
DH = H // NH
LN_EPS = 1e-12
MLC = M * LC

_f32 = jnp.float32
_bf16 = jnp.bfloat16
_CDIM0 = (((0,), (0,)), ((), ()))
_CDIM1 = (((1,), (1,)), ((), ()))


def _dot(a, b):
  return jnp.dot(a, b, preferred_element_type=_f32)


def _gcn_kernel(enc_ref, wr_ref, wrm_ref, wrmm_ref, adj_ref,
                gc1_w_ref, gc1_b_ref, gc2_w_ref, gc2_b_ref,
                proj_w_ref, proj_b_ref,
                cv_ref,
                ohm_s, xs, s1, s2, ms, pooled_s):
  wr = wr_ref[0]
  wrm = wrm_ref[0]
  iota_s = jax.lax.broadcasted_iota(jnp.int32, (S, MLC), 0)
  ohm_s[...] = jnp.where(iota_s == wr,
                         jnp.broadcast_to(wrm, (S, MLC)), 0.0).astype(_bf16)

  xs[...] = jax.lax.dot_general(ohm_s[...], enc_ref[0].astype(_bf16), _CDIM0,
                                preferred_element_type=_f32).astype(_bf16)
  ones = jnp.ones((S, 128), _bf16)
  ms[...] = jax.lax.dot_general(ohm_s[...], ones, _CDIM0,
                                preferred_element_type=_f32)

  s1[...] = _dot(xs[...], gc1_w_ref[...].astype(_bf16)).astype(_bf16)
  b1 = gc1_b_ref[...]
  for m in range(M):
    sl = slice(m * LC, (m + 1) * LC)
    a_m = adj_ref[0, m].astype(_bf16)
    s2[sl, :] = jnp.maximum(_dot(a_m, s1[sl, :]) + b1, 0.0).astype(_bf16)

  s1[...] = _dot(s2[...], gc2_w_ref[...].astype(_bf16)).astype(_bf16)
  b2 = gc2_b_ref[...]
  for m in range(M):
    sl = slice(m * LC, (m + 1) * LC)
    a_m = adj_ref[0, m].astype(_bf16)
    h2m = jnp.maximum(_dot(a_m, s1[sl, :]) + b2, 0.0)
    h2m = h2m * pltpu.repeat(ms[sl, :], H // 128, axis=1)
    xm = xs[sl, :].astype(_f32)
    row = jnp.concatenate([
        jnp.max(xm, axis=0, keepdims=True),
        jnp.max(h2m, axis=0, keepdims=True),
        jnp.sum(xm, axis=0, keepdims=True),
        jnp.sum(h2m, axis=0, keepdims=True),
    ], axis=1)
    pooled_s[m:m + 1, :] = row

  lens = jnp.sum(wrmm_ref[0], axis=1, keepdims=True) + 1e-45
  pooled = pooled_s[...]
  pooled = jnp.concatenate([pooled[:, :2 * H], pooled[:, 2 * H:] / lens],
                           axis=1).astype(_bf16)
  cv_ref[0] = jnp.maximum(
      _dot(pooled, proj_w_ref[...].astype(_bf16)) + proj_b_ref[...], 0.0)


def _attn_kernel(cv_ref, qw_ref, qb_ref, kw_ref, kb_ref, vw_ref, vb_ref,
                 aow_ref, aob_ref, ln1g_ref, ln1b_ref, cnm_ref,
                 attn_ref, qs, ks, vs, ctx_s, sc_s, att_s):
  cv = cv_ref[...]
  rows = cv.shape[0]
  nb = rows // M
  cvb = cv.astype(_bf16)
  qs[...] = (_dot(cvb, qw_ref[...].astype(_bf16)) + qb_ref[...]).astype(_bf16)
  ks[...] = (_dot(cvb, kw_ref[...].astype(_bf16)) + kb_ref[...]).astype(_bf16)
  vs[...] = (_dot(cvb, vw_ref[...].astype(_bf16)) + vb_ref[...]).astype(_bf16)
  scale = 1.0 / math.sqrt(DH)

  for bb in range(nb):
    sl = slice(bb * M, (bb + 1) * M)
    amask = (1.0 - cnm_ref[bb]) * (-10000.0)
    qb_ = qs[sl, :]
    kb_ = ks[sl, :]
    for h in range(NH):
      hs = slice(h * DH, (h + 1) * DH)
      sc = jax.lax.dot_general(qb_[:, hs], kb_[:, hs], _CDIM1,
                               preferred_element_type=_f32) * scale + amask
      i = bb * NH + h
      sc_s[i * M:(i + 1) * M, :] = sc

  s_all = sc_s[...]
  s_all = s_all - jnp.max(s_all, axis=1, keepdims=True)
  e = jnp.exp(s_all)
  att_s[...] = (e / jnp.sum(e, axis=1, keepdims=True)).astype(_bf16)

  for bb in range(nb):
    sl = slice(bb * M, (bb + 1) * M)
    vb_ = vs[sl, :]
    parts = []
    for h in range(NH):
      i = bb * NH + h
      parts.append(_dot(att_s[i * M:(i + 1) * M, :], vb_[:, h * DH:(h + 1) * DH]))
    ctx_s[sl, :] = jnp.concatenate(parts, axis=1).astype(_bf16)

  co = _dot(ctx_s[...], aow_ref[...].astype(_bf16)) + aob_ref[...] + cv
  mu = jnp.mean(co, axis=1, keepdims=True)
  d = co - mu
  var = jnp.mean(d * d, axis=1, keepdims=True)
  attn_ref[...] = (d * jax.lax.rsqrt(var + LN_EPS) * ln1g_ref[...]
                   + ln1b_ref[...])


def _ffn_kernel(attn_ref, intw_ref, intb_ref, outw_ref, outb_ref,
                ln2g_ref, ln2b_ref, decw_ref, decb_ref, tgt_ref, cnmc_ref,
                kl_ref, cn_ref, inter_s):
  attn = attn_ref[...]
  rows = attn.shape[0]
  z = _dot(attn.astype(_bf16), intw_ref[...].astype(_bf16)) + intb_ref[...]
  gelu = z * 0.5 * (1.0 + jax.lax.erf(z * (1.0 / math.sqrt(2.0))))
  inter_s[...] = gelu.astype(_bf16)
  o = _dot(inter_s[...], outw_ref[...].astype(_bf16)) + outb_ref[...] + attn
  mu = jnp.mean(o, axis=1, keepdims=True)
  d = o - mu
  var = jnp.mean(d * d, axis=1, keepdims=True)
  out = d * jax.lax.rsqrt(var + LN_EPS) * ln2g_ref[...] + ln2b_ref[...]
  pred = _dot(out, decw_ref[...]) + decb_ref[...]
  mx = jnp.max(pred, axis=1, keepdims=True)
  e = jnp.exp(pred - mx)
  lse = jnp.log(jnp.sum(e, axis=1, keepdims=True)) + mx
  logp = pred - lse
  t = tgt_ref[...].reshape(rows, NL)
  kl_el = jnp.where(t > 0, t * jnp.log(jnp.where(t > 0, t, 1.0)), 0.0) \
      - t * logp
  klc = jnp.sum(kl_el, axis=1, keepdims=True) * (1.0 / NL)
  cnm = cnmc_ref[...].reshape(rows, 1)
  kl_sum = jnp.sum(klc * cnm, axis=0, keepdims=True)
  cn_sum = jnp.sum(cnm, axis=0, keepdims=True)
  kl_ref[...] = jnp.broadcast_to(kl_sum.reshape(1, 1, 1), (1, 1, 128))
  cn_ref[...] = jnp.broadcast_to(cn_sum.reshape(1, 1, 1), (1, 1, 128))


def _loss_kernel(kl_ref, cn_ref, out_ref):
  kl = jnp.sum(kl_ref[:, 0, 0:1], axis=0, keepdims=True)
  cn = jnp.sum(cn_ref[:, 0, 0:1], axis=0, keepdims=True)
  out_ref[...] = kl / cn


@jax.jit
def kernel(encoder_hs, word_recovery, word_recovery_mask, clause_num_mask,
           adj_matrix, target_labels,
           gc1_w, gc1_b, gc2_w, gc2_b, proj_w, proj_b,
           q_w, q_b, k_w, k_b, v_w, v_b, ao_w, ao_b, ln1_g, ln1_b,
           int_w, int_b, out_w, out_b, ln2_g, ln2_b, dec_w, dec_b):
  wr_flat = word_recovery.reshape(B, 1, MLC)
  wrm_row = word_recovery_mask.astype(_f32).reshape(B, 1, MLC)
  wrm_mat = word_recovery_mask.astype(_f32)
  cnm_row = clause_num_mask.astype(_f32).reshape(B, 1, M)
  cnm_col = clause_num_mask.astype(_f32).reshape(B, M, 1)

  row2 = lambda x: x.reshape(1, -1)
  const2 = lambda b: (0, 0)

  cv = pl.pallas_call(
      _gcn_kernel,
      grid=(B,),
      in_specs=[
          pl.BlockSpec((1, S, H), lambda b: (b, 0, 0)),
          pl.BlockSpec((1, 1, MLC), lambda b: (b, 0, 0)),
          pl.BlockSpec((1, 1, MLC), lambda b: (b, 0, 0)),
          pl.BlockSpec((1, M, LC), lambda b: (b, 0, 0)),
          pl.BlockSpec((1, M, LC, LC), lambda b: (b, 0, 0, 0)),
          pl.BlockSpec((H, H), const2),
          pl.BlockSpec((1, H), const2),
          pl.BlockSpec((H, H), const2),
          pl.BlockSpec((1, H), const2),
          pl.BlockSpec((4 * H, H), const2),
          pl.BlockSpec((1, H), const2),
      ],
      out_specs=pl.BlockSpec((1, M, H), lambda b: (b, 0, 0)),
      out_shape=jax.ShapeDtypeStruct((B, M, H), _f32),
      scratch_shapes=[
          pltpu.VMEM((S, MLC), _bf16),
          pltpu.VMEM((MLC, H), _bf16),
          pltpu.VMEM((MLC, H), _bf16),
          pltpu.VMEM((MLC, H), _bf16),
          pltpu.VMEM((MLC, 128), _f32),
          pltpu.VMEM((M, 4 * H), _f32),
      ],
      compiler_params=pltpu.CompilerParams(
          dimension_semantics=("parallel",),
          vmem_limit_bytes=56 * 1024 * 1024,
      ),
  )(encoder_hs, wr_flat, wrm_row, wrm_mat, adj_matrix,
    gc1_w, row2(gc1_b), gc2_w, row2(gc2_b), proj_w, row2(proj_b))

  cv2 = cv.reshape(B * M, H)
  RB = B // 2
  R = RB * M

  attn = pl.pallas_call(
      _attn_kernel,
      grid=(2,),
      in_specs=[
          pl.BlockSpec((R, H), lambda c: (c, 0)),
          pl.BlockSpec((H, H), const2),
          pl.BlockSpec((1, H), const2),
          pl.BlockSpec((H, H), const2),
          pl.BlockSpec((1, H), const2),
          pl.BlockSpec((H, H), const2),
          pl.BlockSpec((1, H), const2),
          pl.BlockSpec((H, H), const2),
          pl.BlockSpec((1, H), const2),
          pl.BlockSpec((1, H), const2),
          pl.BlockSpec((1, H), const2),
          pl.BlockSpec((RB, 1, M), lambda c: (c, 0, 0)),
      ],
      out_specs=pl.BlockSpec((R, H), lambda c: (c, 0)),
      out_shape=jax.ShapeDtypeStruct((B * M, H), _f32),
      scratch_shapes=[
          pltpu.VMEM((R, H), _bf16),
          pltpu.VMEM((R, H), _bf16),
          pltpu.VMEM((R, H), _bf16),
          pltpu.VMEM((R, H), _bf16),
          pltpu.VMEM((R * NH, M), _f32),
          pltpu.VMEM((R * NH, M), _bf16),
      ],
      compiler_params=pltpu.CompilerParams(
          dimension_semantics=("parallel",),
          vmem_limit_bytes=56 * 1024 * 1024,
      ),
  )(cv2, q_w, row2(q_b), k_w, row2(k_b), v_w, row2(v_b),
    ao_w, row2(ao_b), row2(ln1_g), row2(ln1_b), cnm_row)

  kl_parts, cn_parts = pl.pallas_call(
      _ffn_kernel,
      grid=(2,),
      in_specs=[
          pl.BlockSpec((R, H), lambda c: (c, 0)),
          pl.BlockSpec((H, I), const2),
          pl.BlockSpec((1, I), const2),
          pl.BlockSpec((I, H), const2),
          pl.BlockSpec((1, H), const2),
          pl.BlockSpec((1, H), const2),
          pl.BlockSpec((1, H), const2),
          pl.BlockSpec((H, NL), const2),
          pl.BlockSpec((1, NL), const2),
          pl.BlockSpec((RB, M, NL), lambda c: (c, 0, 0)),
          pl.BlockSpec((RB, M, 1), lambda c: (c, 0, 0)),
      ],
      out_specs=[
          pl.BlockSpec((1, 1, 128), lambda c: (c, 0, 0)),
          pl.BlockSpec((1, 1, 128), lambda c: (c, 0, 0)),
      ],
      out_shape=[
          jax.ShapeDtypeStruct((2, 1, 128), _f32),
          jax.ShapeDtypeStruct((2, 1, 128), _f32),
      ],
      scratch_shapes=[
          pltpu.VMEM((R, I), _bf16),
      ],
      compiler_params=pltpu.CompilerParams(
          dimension_semantics=("parallel",),
          vmem_limit_bytes=56 * 1024 * 1024,
      ),
  )(attn, int_w, row2(int_b), out_w, row2(out_b), row2(ln2_g), row2(ln2_b),
    dec_w, row2(dec_b), target_labels, cnm_col)

  loss = pl.pallas_call(
      _loss_kernel,
      out_shape=jax.ShapeDtypeStruct((1, 1), _f32),
  )(kl_parts, cn_parts)
  return loss.reshape(())


# speedup vs baseline: 1.5446x; 1.1119x over previous
import math

import jax
import jax.numpy as jnp
from jax.experimental import pallas as pl
from jax.experimental.pallas import tpu as pltpu

B, S, H, M, LC, NL, I, NH = 16, 512, 768, 32, 32, 7, 3072, 8
DH = H // NH
LN_EPS = 1e-12
MLC = M * LC

_f32 = jnp.float32
_bf16 = jnp.bfloat16
_CDIM0 = (((0,), (0,)), ((), ()))
_CDIM1 = (((1,), (1,)), ((), ()))


def _dot(a, b):
  return jnp.dot(a, b, preferred_element_type=_f32)


NBK = 2
NR = NBK * MLC
NM = NBK * M


def _gcn_kernel(enc_ref, wr_ref, wrm_ref, wrmm_ref, adj_ref,
                gc1_w_ref, gc1_b_ref, gc2_w_ref, gc2_b_ref,
                proj_w_ref, proj_b_ref,
                cv_ref,
                ohm_s, xs, s1, s2):
  iota_s = jax.lax.broadcasted_iota(jnp.int32, (S, MLC), 0)
  ones = jnp.ones((S, 128), _bf16)
  for c in range(NBK):
    rs = slice(c * MLC, (c + 1) * MLC)
    wr = wr_ref[c]
    wrm = wrm_ref[c]
    ohm_s[...] = jnp.where(iota_s == wr,
                           jnp.broadcast_to(wrm, (S, MLC)), 0.0).astype(_bf16)
    enc_aug = jnp.concatenate([enc_ref[c].astype(_bf16), ones], axis=1)
    xs[rs, :] = jax.lax.dot_general(
        ohm_s[...], enc_aug, _CDIM0,
        preferred_element_type=_f32).astype(_bf16)

  s1[...] = _dot(xs[:, 0:H], gc1_w_ref[...].astype(_bf16)).astype(_bf16)
  b1 = gc1_b_ref[...]
  for c in range(NBK):
    for m in range(M):
      sl = slice(c * MLC + m * LC, c * MLC + (m + 1) * LC)
      a_m = adj_ref[c, m].astype(_bf16)
      s2[sl, :] = jnp.maximum(_dot(a_m, s1[sl, :]) + b1, 0.0).astype(_bf16)

  s1[...] = _dot(s2[...], gc2_w_ref[...].astype(_bf16)).astype(_bf16)
  b2 = gc2_b_ref[...]
  for c in range(NBK):
    for m in range(M):
      sl = slice(c * MLC + m * LC, c * MLC + (m + 1) * LC)
      a_m = adj_ref[c, m].astype(_bf16)
      h2m = jnp.maximum(_dot(a_m, s1[sl, :]) + b2, 0.0).astype(_bf16)
      mrep = pltpu.repeat(xs[sl, H:H + 128], H // 128, axis=1)
      s2[sl, :] = h2m * mrep

  maxx = jnp.max(xs[...].reshape(NM, LC, H + 128), axis=1)[:, 0:H]
  maxh = jnp.max(s2[...].reshape(NM, LC, H), axis=1)
  colgrp = jax.lax.broadcasted_iota(jnp.int32, (NM, NR), 1) // LC
  rowid = jax.lax.broadcasted_iota(jnp.int32, (NM, NR), 0)
  wrm_all = jnp.concatenate([wrm_ref[c] for c in range(NBK)], axis=1)
  pw = jnp.where(colgrp == rowid,
                 jnp.broadcast_to(wrm_all, (NM, NR)), 0.0).astype(_bf16)
  sumx = _dot(pw, xs[:, 0:H])
  sumh = _dot(pw, s2[...])
  lens = jnp.sum(wrmm_ref[...].reshape(NM, LC), axis=1,
                 keepdims=True) + 1e-45
  pooled = jnp.concatenate(
      [maxx.astype(_f32), maxh.astype(_f32), sumx / lens, sumh / lens],
      axis=1).astype(_bf16)
  cv = jnp.maximum(
      _dot(pooled, proj_w_ref[...].astype(_bf16)) + proj_b_ref[...], 0.0)
  cv_ref[...] = cv.reshape(NBK, M, H)


def _tail_kernel(cv_ref, qw_ref, qb_ref, kw_ref, kb_ref, vw_ref, vb_ref,
                 aow_ref, aob_ref, ln1g_ref, ln1b_ref, cnm_ref,
                 intw_hbm, intb_ref, outw_hbm, outb_ref, ln2g_ref, ln2b_ref,
                 decw_ref, decb_ref, tgt_ref, cnmc_ref,
                 loss_ref,
                 qs, ks, vs, ctx_s, sc_s, att_s, intw_v, outw_v, inter_s,
                 sem1, sem2):
  cp1 = pltpu.make_async_copy(intw_hbm, intw_v, sem1)
  cp1.start()
  cp2 = pltpu.make_async_copy(outw_hbm, outw_v, sem2)
  cp2.start()

  cv = cv_ref[...]
  rows = cv.shape[0]
  nb = rows // M
  cvb = cv.astype(_bf16)
  qs[...] = (_dot(cvb, qw_ref[...].astype(_bf16)) + qb_ref[...]).astype(_bf16)
  ks[...] = (_dot(cvb, kw_ref[...].astype(_bf16)) + kb_ref[...]).astype(_bf16)
  vs[...] = (_dot(cvb, vw_ref[...].astype(_bf16)) + vb_ref[...]).astype(_bf16)
  scale = 1.0 / math.sqrt(DH)

  for bb in range(nb):
    sl = slice(bb * M, (bb + 1) * M)
    amask = (1.0 - cnm_ref[bb]) * (-10000.0)
    qb_ = qs[sl, :]
    kb_ = ks[sl, :]
    for h in range(NH):
      hs = slice(h * DH, (h + 1) * DH)
      sc = jax.lax.dot_general(qb_[:, hs], kb_[:, hs], _CDIM1,
                               preferred_element_type=_f32) * scale + amask
      i = bb * NH + h
      sc_s[i * M:(i + 1) * M, :] = sc

  s_all = sc_s[...]
  s_all = s_all - jnp.max(s_all, axis=1, keepdims=True)
  e = jnp.exp(s_all)
  att_s[...] = (e / jnp.sum(e, axis=1, keepdims=True)).astype(_bf16)

  for bb in range(nb):
    sl = slice(bb * M, (bb + 1) * M)
    vb_ = vs[sl, :]
    parts = []
    for h in range(NH):
      i = bb * NH + h
      parts.append(_dot(att_s[i * M:(i + 1) * M, :], vb_[:, h * DH:(h + 1) * DH]))
    ctx_s[sl, :] = jnp.concatenate(parts, axis=1).astype(_bf16)

  co = _dot(ctx_s[...], aow_ref[...].astype(_bf16)) + aob_ref[...] + cv
  mu = jnp.mean(co, axis=1, keepdims=True)
  d = co - mu
  var = jnp.mean(d * d, axis=1, keepdims=True)
  attn = d * jax.lax.rsqrt(var + LN_EPS) * ln1g_ref[...] + ln1b_ref[...]

  cp1.wait()
  z = _dot(attn.astype(_bf16), intw_v[...].astype(_bf16)) + intb_ref[...]
  gelu = z * 0.5 * (1.0 + jax.lax.erf(z * (1.0 / math.sqrt(2.0))))
  inter_s[...] = gelu.astype(_bf16)
  cp2.wait()
  o = _dot(inter_s[...], outw_v[...].astype(_bf16)) + outb_ref[...] + attn
  mu = jnp.mean(o, axis=1, keepdims=True)
  d = o - mu
  var = jnp.mean(d * d, axis=1, keepdims=True)
  out = d * jax.lax.rsqrt(var + LN_EPS) * ln2g_ref[...] + ln2b_ref[...]
  pred = _dot(out, decw_ref[...]) + decb_ref[...]
  mx = jnp.max(pred, axis=1, keepdims=True)
  e = jnp.exp(pred - mx)
  lse = jnp.log(jnp.sum(e, axis=1, keepdims=True)) + mx
  logp = pred - lse
  t = tgt_ref[...].reshape(rows, NL)
  kl_el = jnp.where(t > 0, t * jnp.log(jnp.where(t > 0, t, 1.0)), 0.0) \
      - t * logp
  klc = jnp.sum(kl_el, axis=1, keepdims=True) * (1.0 / NL)
  cnm = cnmc_ref[...].reshape(rows, 1)
  kl_sum = jnp.sum(klc * cnm, axis=0, keepdims=True)
  cn_sum = jnp.sum(cnm, axis=0, keepdims=True)
  loss_ref[...] = kl_sum / cn_sum


@jax.jit
def kernel(encoder_hs, word_recovery, word_recovery_mask, clause_num_mask,
           adj_matrix, target_labels,
           gc1_w, gc1_b, gc2_w, gc2_b, proj_w, proj_b,
           q_w, q_b, k_w, k_b, v_w, v_b, ao_w, ao_b, ln1_g, ln1_b,
           int_w, int_b, out_w, out_b, ln2_g, ln2_b, dec_w, dec_b):
  wr_flat = word_recovery.reshape(B, 1, MLC)
  wrm_row = word_recovery_mask.astype(_f32).reshape(B, 1, MLC)
  wrm_mat = word_recovery_mask.astype(_f32)
  cnm_row = clause_num_mask.astype(_f32).reshape(B, 1, M)
  cnm_col = clause_num_mask.astype(_f32).reshape(B, M, 1)

  row2 = lambda x: x.reshape(1, -1)
  const2 = lambda b: (0, 0)

  cv = pl.pallas_call(
      _gcn_kernel,
      grid=(B // NBK,),
      in_specs=[
          pl.BlockSpec((NBK, S, H), lambda b: (b, 0, 0)),
          pl.BlockSpec((NBK, 1, MLC), lambda b: (b, 0, 0)),
          pl.BlockSpec((NBK, 1, MLC), lambda b: (b, 0, 0)),
          pl.BlockSpec((NBK, M, LC), lambda b: (b, 0, 0)),
          pl.BlockSpec((NBK, M, LC, LC), lambda b: (b, 0, 0, 0)),
          pl.BlockSpec((H, H), const2),
          pl.BlockSpec((1, H), const2),
          pl.BlockSpec((H, H), const2),
          pl.BlockSpec((1, H), const2),
          pl.BlockSpec((4 * H, H), const2),
          pl.BlockSpec((1, H), const2),
      ],
      out_specs=pl.BlockSpec((NBK, M, H), lambda b: (b, 0, 0)),
      out_shape=jax.ShapeDtypeStruct((B, M, H), _f32),
      scratch_shapes=[
          pltpu.VMEM((S, MLC), _bf16),
          pltpu.VMEM((NR, H + 128), _bf16),
          pltpu.VMEM((NR, H), _bf16),
          pltpu.VMEM((NR, H), _bf16),
      ],
      compiler_params=pltpu.CompilerParams(
          dimension_semantics=("parallel",),
          vmem_limit_bytes=56 * 1024 * 1024,
      ),
  )(encoder_hs, wr_flat, wrm_row, wrm_mat, adj_matrix,
    gc1_w, row2(gc1_b), gc2_w, row2(gc2_b), proj_w, row2(proj_b))

  cv2 = cv.reshape(B * M, H)
  R = B * M

  loss = pl.pallas_call(
      _tail_kernel,
      in_specs=[
          pl.BlockSpec((R, H), lambda: (0, 0)),
          pl.BlockSpec((H, H), lambda: (0, 0)),
          pl.BlockSpec((1, H), lambda: (0, 0)),
          pl.BlockSpec((H, H), lambda: (0, 0)),
          pl.BlockSpec((1, H), lambda: (0, 0)),
          pl.BlockSpec((H, H), lambda: (0, 0)),
          pl.BlockSpec((1, H), lambda: (0, 0)),
          pl.BlockSpec((H, H), lambda: (0, 0)),
          pl.BlockSpec((1, H), lambda: (0, 0)),
          pl.BlockSpec((1, H), lambda: (0, 0)),
          pl.BlockSpec((1, H), lambda: (0, 0)),
          pl.BlockSpec((B, 1, M), lambda: (0, 0, 0)),
          pl.BlockSpec(memory_space=pl.ANY),
          pl.BlockSpec((1, I), lambda: (0, 0)),
          pl.BlockSpec(memory_space=pl.ANY),
          pl.BlockSpec((1, H), lambda: (0, 0)),
          pl.BlockSpec((1, H), lambda: (0, 0)),
          pl.BlockSpec((1, H), lambda: (0, 0)),
          pl.BlockSpec((H, NL), lambda: (0, 0)),
          pl.BlockSpec((1, NL), lambda: (0, 0)),
          pl.BlockSpec((B, M, NL), lambda: (0, 0, 0)),
          pl.BlockSpec((B, M, 1), lambda: (0, 0, 0)),
      ],
      out_shape=jax.ShapeDtypeStruct((1, 1), _f32),
      scratch_shapes=[
          pltpu.VMEM((R, H), _bf16),
          pltpu.VMEM((R, H), _bf16),
          pltpu.VMEM((R, H), _bf16),
          pltpu.VMEM((R, H), _bf16),
          pltpu.VMEM((R * NH, M), _f32),
          pltpu.VMEM((R * NH, M), _bf16),
          pltpu.VMEM((H, I), _f32),
          pltpu.VMEM((I, H), _f32),
          pltpu.VMEM((R, I), _bf16),
          pltpu.SemaphoreType.DMA,
          pltpu.SemaphoreType.DMA,
      ],
      compiler_params=pltpu.CompilerParams(
          vmem_limit_bytes=56 * 1024 * 1024,
      ),
  )(cv2, q_w, row2(q_b), k_w, row2(k_b), v_w, row2(v_b),
    ao_w, row2(ao_b), row2(ln1_g), row2(ln1_b), cnm_row,
    int_w, row2(int_b), out_w, row2(out_b), row2(ln2_g), row2(ln2_b),
    dec_w, row2(dec_b), target_labels, cnm_col)
  return loss.reshape(())


# speedup vs baseline: 1.6302x; 1.0554x over previous
import math

import jax
import jax.numpy as jnp
from jax.experimental import pallas as pl
from jax.experimental.pallas import tpu as pltpu

B, S, H, M, LC, NL, I, NH = 16, 512, 768, 32, 32, 7, 3072, 8
DH = H // NH
LN_EPS = 1e-12
MLC = M * LC

_f32 = jnp.float32
_bf16 = jnp.bfloat16
_CDIM0 = (((0,), (0,)), ((), ()))
_CDIM1 = (((1,), (1,)), ((), ()))


def _dot(a, b):
  return jnp.dot(a, b, preferred_element_type=_f32)


NBK = 2
NR = NBK * MLC
NM = NBK * M


def _gcn_kernel(enc_ref, wr_ref, wrm_ref, wrmm_ref, adj_ref,
                gc1_w_ref, gc1_b_ref, gc2_w_ref, gc2_b_ref,
                proj_w_ref, proj_b_ref,
                cv_ref,
                ohm_s, xs, s1, s2, bd_s):
  iota_s = jax.lax.broadcasted_iota(jnp.int32, (S, MLC), 0)
  ones = jnp.ones((S, 128), _bf16)
  for c in range(NBK):
    rs = slice(c * MLC, (c + 1) * MLC)
    wr = wr_ref[c]
    wrm = wrm_ref[c]
    ohm_s[...] = jnp.where(iota_s == wr,
                           jnp.broadcast_to(wrm, (S, MLC)), 0.0).astype(_bf16)
    enc_aug = jnp.concatenate([enc_ref[c].astype(_bf16), ones], axis=1)
    xs[rs, :] = jax.lax.dot_general(
        ohm_s[...], enc_aug, _CDIM0,
        preferred_element_type=_f32).astype(_bf16)

  GC = 4
  GR = GC * LC
  NG = M // GC
  rowgrp = jax.lax.broadcasted_iota(jnp.int32, (GR, GR), 0) // LC
  colgrp = jax.lax.broadcasted_iota(jnp.int32, (GR, GR), 1) // LC
  for c in range(NBK):
    for g in range(NG):
      a4 = adj_ref[c, GC * g:GC * (g + 1)].reshape(GR, LC)
      tiled = jnp.concatenate([a4, a4, a4, a4], axis=1)
      bd = jnp.where(rowgrp == colgrp, tiled, 0.0).astype(_bf16)
      i = c * NG + g
      bd_s[i * GR:(i + 1) * GR, :] = bd

  s1[...] = _dot(xs[:, 0:H], gc1_w_ref[...].astype(_bf16)).astype(_bf16)
  b1 = gc1_b_ref[...]
  for i in range(NBK * NG):
    sl = slice(i * GR, (i + 1) * GR)
    s2[sl, :] = jnp.maximum(
        _dot(bd_s[sl, :], s1[sl, :]) + b1, 0.0).astype(_bf16)

  s1[...] = _dot(s2[...], gc2_w_ref[...].astype(_bf16)).astype(_bf16)
  b2 = gc2_b_ref[...]
  for i in range(NBK * NG):
    sl = slice(i * GR, (i + 1) * GR)
    h2g = jnp.maximum(_dot(bd_s[sl, :], s1[sl, :]) + b2, 0.0).astype(_bf16)
    mrep = pltpu.repeat(xs[sl, H:H + 128], H // 128, axis=1)
    s2[sl, :] = h2g * mrep

  maxx = jnp.max(xs[:, 0:H].reshape(NM, LC, H), axis=1)
  maxh = jnp.max(s2[...].reshape(NM, LC, H), axis=1)
  colgrp = jax.lax.broadcasted_iota(jnp.int32, (NM, NR), 1) // LC
  rowid = jax.lax.broadcasted_iota(jnp.int32, (NM, NR), 0)
  wrm_all = jnp.concatenate([wrm_ref[c] for c in range(NBK)], axis=1)
  pw = jnp.where(colgrp == rowid,
                 jnp.broadcast_to(wrm_all, (NM, NR)), 0.0).astype(_bf16)
  sumx = _dot(pw, xs[:, 0:H])
  sumh = _dot(pw, s2[...])
  lens = jnp.sum(wrmm_ref[...].reshape(NM, LC), axis=1,
                 keepdims=True) + 1e-45
  pooled = jnp.concatenate(
      [maxx.astype(_f32), maxh.astype(_f32), sumx / lens, sumh / lens],
      axis=1).astype(_bf16)
  cv = jnp.maximum(
      _dot(pooled, proj_w_ref[...].astype(_bf16)) + proj_b_ref[...], 0.0)
  cv_ref[...] = cv.reshape(NBK, M, H)


def _tail_kernel(cv_ref, qw_ref, qb_ref, kw_ref, kb_ref, vw_ref, vb_ref,
                 aow_ref, aob_ref, ln1g_ref, ln1b_ref, cnm_ref,
                 intw_hbm, intb_ref, outw_hbm, outb_ref, ln2g_ref, ln2b_ref,
                 decw_ref, decb_ref, tgt_ref, cnmc_ref,
                 loss_ref,
                 qs, ks, vs, ctx_s, sc_s, att_s, intw_v, outw_v, inter_s,
                 sem1, sem2):
  cp1 = pltpu.make_async_copy(intw_hbm, intw_v, sem1)
  cp1.start()
  cp2 = pltpu.make_async_copy(outw_hbm, outw_v, sem2)
  cp2.start()

  cv = cv_ref[...]
  rows = cv.shape[0]
  nb = rows // M
  cvb = cv.astype(_bf16)
  qs[...] = (_dot(cvb, qw_ref[...].astype(_bf16)) + qb_ref[...]).astype(_bf16)
  ks[...] = (_dot(cvb, kw_ref[...].astype(_bf16)) + kb_ref[...]).astype(_bf16)
  vs[...] = (_dot(cvb, vw_ref[...].astype(_bf16)) + vb_ref[...]).astype(_bf16)
  scale = 1.0 / math.sqrt(DH)

  for bb in range(nb):
    sl = slice(bb * M, (bb + 1) * M)
    amask = (1.0 - cnm_ref[bb]) * (-10000.0)
    qb_ = qs[sl, :]
    kb_ = ks[sl, :]
    for h in range(NH):
      hs = slice(h * DH, (h + 1) * DH)
      sc = jax.lax.dot_general(qb_[:, hs], kb_[:, hs], _CDIM1,
                               preferred_element_type=_f32) * scale + amask
      i = bb * NH + h
      sc_s[i * M:(i + 1) * M, :] = sc

  s_all = sc_s[...]
  s_all = s_all - jnp.max(s_all, axis=1, keepdims=True)
  e = jnp.exp(s_all)
  att_s[...] = (e / jnp.sum(e, axis=1, keepdims=True)).astype(_bf16)

  for bb in range(nb):
    sl = slice(bb * M, (bb + 1) * M)
    vb_ = vs[sl, :]
    parts = []
    for h in range(NH):
      i = bb * NH + h
      parts.append(_dot(att_s[i * M:(i + 1) * M, :], vb_[:, h * DH:(h + 1) * DH]))
    ctx_s[sl, :] = jnp.concatenate(parts, axis=1).astype(_bf16)

  co = _dot(ctx_s[...], aow_ref[...].astype(_bf16)) + aob_ref[...] + cv
  mu = jnp.mean(co, axis=1, keepdims=True)
  d = co - mu
  var = jnp.mean(d * d, axis=1, keepdims=True)
  attn = d * jax.lax.rsqrt(var + LN_EPS) * ln1g_ref[...] + ln1b_ref[...]

  cp1.wait()
  z = _dot(attn.astype(_bf16), intw_v[...].astype(_bf16)) + intb_ref[...]
  gelu = z * 0.5 * (1.0 + jax.lax.erf(z * (1.0 / math.sqrt(2.0))))
  inter_s[...] = gelu.astype(_bf16)
  cp2.wait()
  o = _dot(inter_s[...], outw_v[...].astype(_bf16)) + outb_ref[...] + attn
  mu = jnp.mean(o, axis=1, keepdims=True)
  d = o - mu
  var = jnp.mean(d * d, axis=1, keepdims=True)
  out = d * jax.lax.rsqrt(var + LN_EPS) * ln2g_ref[...] + ln2b_ref[...]
  pred = _dot(out, decw_ref[...]) + decb_ref[...]
  mx = jnp.max(pred, axis=1, keepdims=True)
  e = jnp.exp(pred - mx)
  lse = jnp.log(jnp.sum(e, axis=1, keepdims=True)) + mx
  logp = pred - lse
  t = tgt_ref[...].reshape(rows, NL)
  kl_el = jnp.where(t > 0, t * jnp.log(jnp.where(t > 0, t, 1.0)), 0.0) \
      - t * logp
  klc = jnp.sum(kl_el, axis=1, keepdims=True) * (1.0 / NL)
  cnm = cnmc_ref[...].reshape(rows, 1)
  kl_sum = jnp.sum(klc * cnm, axis=0, keepdims=True)
  cn_sum = jnp.sum(cnm, axis=0, keepdims=True)
  loss_ref[...] = kl_sum / cn_sum


@jax.jit
def kernel(encoder_hs, word_recovery, word_recovery_mask, clause_num_mask,
           adj_matrix, target_labels,
           gc1_w, gc1_b, gc2_w, gc2_b, proj_w, proj_b,
           q_w, q_b, k_w, k_b, v_w, v_b, ao_w, ao_b, ln1_g, ln1_b,
           int_w, int_b, out_w, out_b, ln2_g, ln2_b, dec_w, dec_b):
  wr_flat = word_recovery.reshape(B, 1, MLC)
  wrm_row = word_recovery_mask.astype(_f32).reshape(B, 1, MLC)
  wrm_mat = word_recovery_mask.astype(_f32)
  cnm_row = clause_num_mask.astype(_f32).reshape(B, 1, M)
  cnm_col = clause_num_mask.astype(_f32).reshape(B, M, 1)

  row2 = lambda x: x.reshape(1, -1)
  const2 = lambda b: (0, 0)

  cv = pl.pallas_call(
      _gcn_kernel,
      grid=(B // NBK,),
      in_specs=[
          pl.BlockSpec((NBK, S, H), lambda b: (b, 0, 0)),
          pl.BlockSpec((NBK, 1, MLC), lambda b: (b, 0, 0)),
          pl.BlockSpec((NBK, 1, MLC), lambda b: (b, 0, 0)),
          pl.BlockSpec((NBK, M, LC), lambda b: (b, 0, 0)),
          pl.BlockSpec((NBK, M, LC, LC), lambda b: (b, 0, 0, 0)),
          pl.BlockSpec((H, H), const2),
          pl.BlockSpec((1, H), const2),
          pl.BlockSpec((H, H), const2),
          pl.BlockSpec((1, H), const2),
          pl.BlockSpec((4 * H, H), const2),
          pl.BlockSpec((1, H), const2),
      ],
      out_specs=pl.BlockSpec((NBK, M, H), lambda b: (b, 0, 0)),
      out_shape=jax.ShapeDtypeStruct((B, M, H), _f32),
      scratch_shapes=[
          pltpu.VMEM((S, MLC), _bf16),
          pltpu.VMEM((NR, H + 128), _bf16),
          pltpu.VMEM((NR, H), _bf16),
          pltpu.VMEM((NR, H), _bf16),
          pltpu.VMEM((NR, 128), _bf16),
      ],
      compiler_params=pltpu.CompilerParams(
          dimension_semantics=("parallel",),
          vmem_limit_bytes=56 * 1024 * 1024,
      ),
  )(encoder_hs, wr_flat, wrm_row, wrm_mat, adj_matrix,
    gc1_w, row2(gc1_b), gc2_w, row2(gc2_b), proj_w, row2(proj_b))

  cv2 = cv.reshape(B * M, H)
  R = B * M

  loss = pl.pallas_call(
      _tail_kernel,
      in_specs=[
          pl.BlockSpec((R, H), lambda: (0, 0)),
          pl.BlockSpec((H, H), lambda: (0, 0)),
          pl.BlockSpec((1, H), lambda: (0, 0)),
          pl.BlockSpec((H, H), lambda: (0, 0)),
          pl.BlockSpec((1, H), lambda: (0, 0)),
          pl.BlockSpec((H, H), lambda: (0, 0)),
          pl.BlockSpec((1, H), lambda: (0, 0)),
          pl.BlockSpec((H, H), lambda: (0, 0)),
          pl.BlockSpec((1, H), lambda: (0, 0)),
          pl.BlockSpec((1, H), lambda: (0, 0)),
          pl.BlockSpec((1, H), lambda: (0, 0)),
          pl.BlockSpec((B, 1, M), lambda: (0, 0, 0)),
          pl.BlockSpec(memory_space=pl.ANY),
          pl.BlockSpec((1, I), lambda: (0, 0)),
          pl.BlockSpec(memory_space=pl.ANY),
          pl.BlockSpec((1, H), lambda: (0, 0)),
          pl.BlockSpec((1, H), lambda: (0, 0)),
          pl.BlockSpec((1, H), lambda: (0, 0)),
          pl.BlockSpec((H, NL), lambda: (0, 0)),
          pl.BlockSpec((1, NL), lambda: (0, 0)),
          pl.BlockSpec((B, M, NL), lambda: (0, 0, 0)),
          pl.BlockSpec((B, M, 1), lambda: (0, 0, 0)),
      ],
      out_shape=jax.ShapeDtypeStruct((1, 1), _f32),
      scratch_shapes=[
          pltpu.VMEM((R, H), _bf16),
          pltpu.VMEM((R, H), _bf16),
          pltpu.VMEM((R, H), _bf16),
          pltpu.VMEM((R, H), _bf16),
          pltpu.VMEM((R * NH, M), _f32),
          pltpu.VMEM((R * NH, M), _bf16),
          pltpu.VMEM((H, I), _f32),
          pltpu.VMEM((I, H), _f32),
          pltpu.VMEM((R, I), _bf16),
          pltpu.SemaphoreType.DMA,
          pltpu.SemaphoreType.DMA,
      ],
      compiler_params=pltpu.CompilerParams(
          vmem_limit_bytes=56 * 1024 * 1024,
      ),
  )(cv2, q_w, row2(q_b), k_w, row2(k_b), v_w, row2(v_b),
    ao_w, row2(ao_b), row2(ln1_g), row2(ln1_b), cnm_row,
    int_w, row2(int_b), out_w, row2(out_b), row2(ln2_g), row2(ln2_b),
    dec_w, row2(dec_b), target_labels, cnm_col)
  return loss.reshape(())
